# baseline (speedup 1.0000x reference)
import functools

import jax
import jax.numpy as jnp
from jax import lax
from jax.experimental import pallas as pl
from jax.experimental.pallas import tpu as pltpu
from jax.experimental.pallas import tpu_sc as plsc

_N_ROWS = 4096
_N_COLS = 4096
_TOTAL = _N_ROWS * _N_COLS

_SC_ROWS = 4096
_TC_ROWS = _N_ROWS - _SC_ROWS

_NC, _NS = 2, 16
_NW = _NC * _NS
_CHUNK = 16384
_UNROLL = 8
_NACC = 4
_SC_ELEMS = _SC_ROWS * _N_COLS
_PER_W = _SC_ELEMS // _NW
_NCHUNK = _PER_W // _CHUNK

_LN2 = 0.6931471805599453
_C0 = 0.0009250321113061788 - 127.0 * _LN2
_C1 = 0.9797534129748476
_C2 = -0.39353580230192053
_C3 = 0.10668473260369084


def _focal_term(xv, tv):
    yv = 1.0 - xv
    msk = tv != 0
    p = jnp.where(msk, xv, yv)
    om = jnp.where(msk, yv, xv)
    bits = lax.bitcast_convert_type(p, jnp.int32)
    e_f = (bits >> 23).astype(jnp.float32)
    m = lax.bitcast_convert_type((bits & 0x007FFFFF) | 0x3F800000, jnp.float32)
    r = m - 1.0
    poly = _C0 + r * (_C1 + r * (_C2 + r * _C3))
    ln_p = e_f * _LN2 + poly
    return om * om * ln_p


def _sc_focal_body(x_hbm, t_hbm, out_hbm, xb, tb, accb, s0, s1, s2, s3):
    wid = lax.axis_index("s") * _NC + lax.axis_index("c")
    base = wid * _PER_W
    sems = (s0, s1, s2, s3)

    def start(c):
        slot = c % 2
        cx = pltpu.async_copy(
            x_hbm.at[pl.ds(base + c * _CHUNK, _CHUNK)], xb.at[slot], sems[slot])
        ct = pltpu.async_copy(
            t_hbm.at[pl.ds(base + c * _CHUNK, _CHUNK)], tb.at[slot], sems[2 + slot])
        return cx, ct

    def compute(slot, acc):
        @plsc.parallel_loop(0, _CHUNK, 16, unroll=_UNROLL, carry=acc)
        def final(i, a):
            xv = xb[slot, pl.ds(i, 16)]
            tv = tb[slot, pl.ds(i, 16)]
            return a - _focal_term(xv, tv)
        return final

    handles = {0: start(0)}
    if _NCHUNK > 1:
        handles[1] = start(1)
    acc = jnp.zeros((16,), jnp.float32)
    for c in range(_NCHUNK):
        cx, ct = handles.pop(c)
        cx.wait()
        ct.wait()
        acc = compute(c % 2, acc)
        if c + 2 < _NCHUNK:
            handles[c + 2] = start(c + 2)
    accb[...] = acc
    pltpu.sync_copy(accb, out_hbm.at[wid])


_sc_focal = functools.partial(
    pl.kernel,
    out_type=jax.ShapeDtypeStruct((_NW, 16), jnp.float32),
    mesh=plsc.VectorSubcoreMesh(core_axis_name="c", subcore_axis_name="s"),
    scratch_types=[
        pltpu.VMEM((2, _CHUNK), jnp.float32),
        pltpu.VMEM((2, _CHUNK), jnp.int32),
        pltpu.VMEM((16,), jnp.float32),
        pltpu.SemaphoreType.DMA,
        pltpu.SemaphoreType.DMA,
        pltpu.SemaphoreType.DMA,
        pltpu.SemaphoreType.DMA,
    ],
)(_sc_focal_body)


_TC_BLOCK_ROWS = 256


def _tc_focal_body(x_ref, t_ref, out_ref):
    i = pl.program_id(0)
    x = x_ref[...]
    t = t_ref[...]
    p = jnp.where(t != 0, x, 1.0 - x)
    one_m = 1.0 - p
    s = -jnp.sum(one_m * one_m * jnp.log(p))

    @pl.when(i == 0)
    def _init():
        out_ref[0, 0] = s

    @pl.when(i != 0)
    def _acc():
        out_ref[0, 0] += s


def _tc_focal(x, t):
    grid = _TC_ROWS // _TC_BLOCK_ROWS
    return pl.pallas_call(
        _tc_focal_body,
        grid=(grid,),
        in_specs=[
            pl.BlockSpec((_TC_BLOCK_ROWS, _N_COLS), lambda i: (i, 0)),
            pl.BlockSpec((_TC_BLOCK_ROWS, _N_COLS), lambda i: (i, 0)),
        ],
        out_specs=pl.BlockSpec(memory_space=pltpu.SMEM),
        out_shape=jax.ShapeDtypeStruct((1, 1), jnp.float32),
    )(x, t)


def kernel(inputs, targets):
    total = jnp.float32(0.0)
    if _SC_ROWS > 0:
        x_flat = inputs[:_SC_ROWS].reshape(-1)
        t_flat = targets[:_SC_ROWS].reshape(-1)
        total = total + jnp.sum(_sc_focal(x_flat, t_flat))
    if _TC_ROWS > 0:
        total = total + _tc_focal(inputs[_SC_ROWS:], targets[_SC_ROWS:])[0, 0]
    return total / _TOTAL


# speedup vs baseline: 1.1869x; 1.1869x over previous
import functools

import jax
import jax.numpy as jnp
from jax import lax
from jax.experimental import pallas as pl
from jax.experimental.pallas import tpu as pltpu
from jax.experimental.pallas import tpu_sc as plsc

_N_ROWS = 4096
_N_COLS = 4096
_TOTAL = _N_ROWS * _N_COLS

_SC_ROWS = 4096
_TC_ROWS = _N_ROWS - _SC_ROWS

_NC, _NS = 2, 16
_NW = _NC * _NS
_CHUNK = 16384
_UNROLL = 8
_NACC = 4
_SC_ELEMS = _SC_ROWS * _N_COLS
_PER_W = _SC_ELEMS // _NW
_NCHUNK = _PER_W // _CHUNK

_LN2 = 0.6931471805599453
_BSCALE = _LN2 / (1 << 23)
_C0 = 0.0009250321113061788 - 127.0 * _LN2
_C1 = 0.9797534129748476 - _LN2
_C2 = -0.39353580230192053
_C3 = 0.10668473260369084


def _focal_term(xv, tv):
    yv = 1.0 - xv
    p = jnp.where(tv != 0, xv, yv)
    om = 1.0 - p
    bits = lax.bitcast_convert_type(p, jnp.int32)
    bf = bits.astype(jnp.float32)
    m = lax.bitcast_convert_type((bits & 0x007FFFFF) | 0x3F800000, jnp.float32)
    r = m - 1.0
    poly = _C0 + r * (_C1 + r * (_C2 + r * _C3))
    ln_p = bf * _BSCALE + poly
    return om * om * ln_p


def _sc_focal_body(x_hbm, t_hbm, out_hbm, xb, tb, accb, s0, s1, s2, s3):
    wid = lax.axis_index("s") * _NC + lax.axis_index("c")
    base = wid * _PER_W
    sems = (s0, s1, s2, s3)

    def start(c):
        slot = c % 2
        cx = pltpu.async_copy(
            x_hbm.at[pl.ds(base + c * _CHUNK, _CHUNK)], xb.at[slot], sems[slot])
        ct = pltpu.async_copy(
            t_hbm.at[pl.ds(base + c * _CHUNK, _CHUNK)], tb.at[slot], sems[2 + slot])
        return cx, ct

    def compute(slot, accs):
        def body(i, accs):
            off = i * (16 * _UNROLL)
            out = []
            for u in range(_UNROLL):
                xv = xb[slot, pl.ds(off + u * 16, 16)]
                tv = tb[slot, pl.ds(off + u * 16, 16)]
                out.append(accs[u] - _focal_term(xv, tv))
            return tuple(out)
        return lax.fori_loop(0, _CHUNK // (16 * _UNROLL), body, accs)

    handles = {0: start(0)}
    if _NCHUNK > 1:
        handles[1] = start(1)
    accs = tuple(jnp.zeros((16,), jnp.float32) for _ in range(_UNROLL))
    for c in range(_NCHUNK):
        cx, ct = handles.pop(c)
        cx.wait()
        ct.wait()
        accs = compute(c % 2, accs)
        if c + 2 < _NCHUNK:
            handles[c + 2] = start(c + 2)
    acc = accs[0]
    for u in range(1, _UNROLL):
        acc = acc + accs[u]
    accb[...] = acc
    pltpu.sync_copy(accb, out_hbm.at[wid])


_sc_focal = functools.partial(
    pl.kernel,
    out_type=jax.ShapeDtypeStruct((_NW, 16), jnp.float32),
    mesh=plsc.VectorSubcoreMesh(core_axis_name="c", subcore_axis_name="s"),
    scratch_types=[
        pltpu.VMEM((2, _CHUNK), jnp.float32),
        pltpu.VMEM((2, _CHUNK), jnp.int32),
        pltpu.VMEM((16,), jnp.float32),
        pltpu.SemaphoreType.DMA,
        pltpu.SemaphoreType.DMA,
        pltpu.SemaphoreType.DMA,
        pltpu.SemaphoreType.DMA,
    ],
)(_sc_focal_body)


_TC_BLOCK_ROWS = 256


def _tc_focal_body(x_ref, t_ref, out_ref):
    i = pl.program_id(0)
    x = x_ref[...]
    t = t_ref[...]
    p = jnp.where(t != 0, x, 1.0 - x)
    one_m = 1.0 - p
    s = -jnp.sum(one_m * one_m * jnp.log(p))

    @pl.when(i == 0)
    def _init():
        out_ref[0, 0] = s

    @pl.when(i != 0)
    def _acc():
        out_ref[0, 0] += s


def _tc_focal(x, t):
    grid = _TC_ROWS // _TC_BLOCK_ROWS
    return pl.pallas_call(
        _tc_focal_body,
        grid=(grid,),
        in_specs=[
            pl.BlockSpec((_TC_BLOCK_ROWS, _N_COLS), lambda i: (i, 0)),
            pl.BlockSpec((_TC_BLOCK_ROWS, _N_COLS), lambda i: (i, 0)),
        ],
        out_specs=pl.BlockSpec(memory_space=pltpu.SMEM),
        out_shape=jax.ShapeDtypeStruct((1, 1), jnp.float32),
    )(x, t)


def kernel(inputs, targets):
    total = jnp.float32(0.0)
    if _SC_ROWS > 0:
        x_flat = inputs[:_SC_ROWS].reshape(-1)
        t_flat = targets[:_SC_ROWS].reshape(-1)
        total = total + jnp.sum(_sc_focal(x_flat, t_flat))
    if _TC_ROWS > 0:
        total = total + _tc_focal(inputs[_SC_ROWS:], targets[_SC_ROWS:])[0, 0]
    return total / _TOTAL


# speedup vs baseline: 2.0754x; 1.7485x over previous
import functools

import jax
import jax.numpy as jnp
from jax import lax
from jax.experimental import pallas as pl
from jax.experimental.pallas import tpu as pltpu
from jax.experimental.pallas import tpu_sc as plsc

_N_ROWS = 4096
_N_COLS = 4096
_TOTAL = _N_ROWS * _N_COLS

_SC_ROWS = 512
_TC_ROWS = _N_ROWS - _SC_ROWS

_NC, _NS = 2, 16
_NW = _NC * _NS
_CHUNK = 16384
_UNROLL = 8
_NACC = 4
_SC_ELEMS = _SC_ROWS * _N_COLS
_PER_W = _SC_ELEMS // _NW
_NCHUNK = _PER_W // _CHUNK

_LN2 = 0.6931471805599453
_BSCALE = _LN2 / (1 << 23)
_C0 = 0.0009250321113061788 - 127.0 * _LN2
_C1 = 0.9797534129748476 - _LN2
_C2 = -0.39353580230192053
_C3 = 0.10668473260369084


def _focal_term(xv, tv):
    yv = 1.0 - xv
    p = jnp.where(tv != 0, xv, yv)
    om = 1.0 - p
    bits = lax.bitcast_convert_type(p, jnp.int32)
    bf = bits.astype(jnp.float32)
    m = lax.bitcast_convert_type((bits & 0x007FFFFF) | 0x3F800000, jnp.float32)
    r = m - 1.0
    poly = _C0 + r * (_C1 + r * (_C2 + r * _C3))
    ln_p = bf * _BSCALE + poly
    return om * om * ln_p


def _sc_focal_body(x_hbm, t_hbm, out_hbm, xb, tb, accb, s0, s1, s2, s3):
    wid = lax.axis_index("s") * _NC + lax.axis_index("c")
    base = wid * _PER_W
    sems = (s0, s1, s2, s3)

    def start(c):
        slot = c % 2
        cx = pltpu.async_copy(
            x_hbm.at[pl.ds(base + c * _CHUNK, _CHUNK)], xb.at[slot], sems[slot])
        ct = pltpu.async_copy(
            t_hbm.at[pl.ds(base + c * _CHUNK, _CHUNK)], tb.at[slot], sems[2 + slot])
        return cx, ct

    def compute(slot, accs):
        def body(i, accs):
            off = i * (16 * _UNROLL)
            out = []
            for u in range(_UNROLL):
                xv = xb[slot, pl.ds(off + u * 16, 16)]
                tv = tb[slot, pl.ds(off + u * 16, 16)]
                out.append(accs[u] - _focal_term(xv, tv))
            return tuple(out)
        return lax.fori_loop(0, _CHUNK // (16 * _UNROLL), body, accs)

    handles = {0: start(0)}
    if _NCHUNK > 1:
        handles[1] = start(1)
    accs = tuple(jnp.zeros((16,), jnp.float32) for _ in range(_UNROLL))
    for c in range(_NCHUNK):
        cx, ct = handles.pop(c)
        cx.wait()
        ct.wait()
        accs = compute(c % 2, accs)
        if c + 2 < _NCHUNK:
            handles[c + 2] = start(c + 2)
    acc = accs[0]
    for u in range(1, _UNROLL):
        acc = acc + accs[u]
    accb[...] = acc
    pltpu.sync_copy(accb, out_hbm.at[wid])


_sc_focal = functools.partial(
    pl.kernel,
    out_type=jax.ShapeDtypeStruct((_NW, 16), jnp.float32),
    mesh=plsc.VectorSubcoreMesh(core_axis_name="c", subcore_axis_name="s"),
    scratch_types=[
        pltpu.VMEM((2, _CHUNK), jnp.float32),
        pltpu.VMEM((2, _CHUNK), jnp.int32),
        pltpu.VMEM((16,), jnp.float32),
        pltpu.SemaphoreType.DMA,
        pltpu.SemaphoreType.DMA,
        pltpu.SemaphoreType.DMA,
        pltpu.SemaphoreType.DMA,
    ],
)(_sc_focal_body)


_TC_BLOCK_ROWS = 256


def _tc_focal_body(x_ref, t_ref, out_ref):
    i = pl.program_id(0)
    x = x_ref[...]
    t = t_ref[...]
    p = jnp.where(t != 0, x, 1.0 - x)
    one_m = 1.0 - p
    s = -jnp.sum(one_m * one_m * jnp.log(p))

    @pl.when(i == 0)
    def _init():
        out_ref[0, 0] = s

    @pl.when(i != 0)
    def _acc():
        out_ref[0, 0] += s


def _tc_focal(x, t):
    grid = _TC_ROWS // _TC_BLOCK_ROWS
    return pl.pallas_call(
        _tc_focal_body,
        grid=(grid,),
        in_specs=[
            pl.BlockSpec((_TC_BLOCK_ROWS, _N_COLS), lambda i: (i, 0)),
            pl.BlockSpec((_TC_BLOCK_ROWS, _N_COLS), lambda i: (i, 0)),
        ],
        out_specs=pl.BlockSpec(memory_space=pltpu.SMEM),
        out_shape=jax.ShapeDtypeStruct((1, 1), jnp.float32),
    )(x, t)


def kernel(inputs, targets):
    total = jnp.float32(0.0)
    if _SC_ROWS > 0:
        x_flat = inputs[:_SC_ROWS].reshape(-1)
        t_flat = targets[:_SC_ROWS].reshape(-1)
        total = total + jnp.sum(_sc_focal(x_flat, t_flat))
    if _TC_ROWS > 0:
        total = total + _tc_focal(inputs[_SC_ROWS:], targets[_SC_ROWS:])[0, 0]
    return total / _TOTAL


# speedup vs baseline: 5.4896x; 2.6451x over previous
import functools

import jax
import jax.numpy as jnp
from jax import lax
from jax.experimental import pallas as pl
from jax.experimental.pallas import tpu as pltpu
from jax.experimental.pallas import tpu_sc as plsc

_N_ROWS = 4096
_N_COLS = 4096
_TOTAL = _N_ROWS * _N_COLS

_SC_ROWS = 512
_TC_ROWS = _N_ROWS - _SC_ROWS

_NC, _NS = 2, 16
_NW = _NC * _NS
_CROWS = 4
_UNROLL = 8
_RPW = _SC_ROWS // _NW
_NCHUNK = _RPW // _CROWS

_LN2 = 0.6931471805599453
_BSCALE = _LN2 / (1 << 23)
_C0 = 0.0009250321113061788 - 127.0 * _LN2
_C1 = 0.9797534129748476 - _LN2
_C2 = -0.39353580230192053
_C3 = 0.10668473260369084


def _focal_term(xv, tv):
    yv = 1.0 - xv
    p = jnp.where(tv != 0, xv, yv)
    om = 1.0 - p
    bits = lax.bitcast_convert_type(p, jnp.int32)
    bf = bits.astype(jnp.float32)
    m = lax.bitcast_convert_type((bits & 0x007FFFFF) | 0x3F800000, jnp.float32)
    r = m - 1.0
    poly = _C0 + r * (_C1 + r * (_C2 + r * _C3))
    ln_p = bf * _BSCALE + poly
    return om * om * ln_p


def _sc_focal_body(x_hbm, t_hbm, out_hbm, xb, tb, accb, s0, s1, s2, s3):
    wid = lax.axis_index("s") * _NC + lax.axis_index("c")
    row0 = wid * _RPW
    sems = (s0, s1, s2, s3)

    def start(c):
        slot = c % 2
        rows = pl.ds(row0 + c * _CROWS, _CROWS)
        cx = pltpu.async_copy(x_hbm.at[rows], xb.at[slot], sems[slot])
        ct = pltpu.async_copy(t_hbm.at[rows], tb.at[slot], sems[2 + slot])
        return cx, ct

    def compute(slot, accs):
        for rr in range(_CROWS):
            def body(i, accs):
                off = i * (16 * _UNROLL)
                out = []
                for u in range(_UNROLL):
                    xv = xb[slot, rr, pl.ds(off + u * 16, 16)]
                    tv = tb[slot, rr, pl.ds(off + u * 16, 16)]
                    out.append(accs[u] - _focal_term(xv, tv))
                return tuple(out)
            accs = lax.fori_loop(0, _N_COLS // (16 * _UNROLL), body, accs)
        return accs

    handles = {0: start(0)}
    if _NCHUNK > 1:
        handles[1] = start(1)
    accs = tuple(jnp.zeros((16,), jnp.float32) for _ in range(_UNROLL))
    for c in range(_NCHUNK):
        cx, ct = handles.pop(c)
        cx.wait()
        ct.wait()
        accs = compute(c % 2, accs)
        if c + 2 < _NCHUNK:
            handles[c + 2] = start(c + 2)
    acc = accs[0]
    for u in range(1, _UNROLL):
        acc = acc + accs[u]
    accb[...] = acc
    pltpu.sync_copy(accb, out_hbm.at[wid])


_sc_focal = functools.partial(
    pl.kernel,
    out_type=jax.ShapeDtypeStruct((_NW, 16), jnp.float32),
    mesh=plsc.VectorSubcoreMesh(core_axis_name="c", subcore_axis_name="s"),
    scratch_types=[
        pltpu.VMEM((2, _CROWS, _N_COLS), jnp.float32),
        pltpu.VMEM((2, _CROWS, _N_COLS), jnp.int32),
        pltpu.VMEM((16,), jnp.float32),
        pltpu.SemaphoreType.DMA,
        pltpu.SemaphoreType.DMA,
        pltpu.SemaphoreType.DMA,
        pltpu.SemaphoreType.DMA,
    ],
)(_sc_focal_body)


_TC_BLOCK_ROWS = 256
_TC_BLOCK_OFF = _SC_ROWS // _TC_BLOCK_ROWS


def _tc_focal_body(x_ref, t_ref, out_ref):
    i = pl.program_id(0)
    x = x_ref[...]
    t = t_ref[...]
    p = jnp.where(t != 0, x, 1.0 - x)
    one_m = 1.0 - p
    s = -jnp.sum(one_m * one_m * jnp.log(p))

    @pl.when(i == 0)
    def _init():
        out_ref[0, 0] = s

    @pl.when(i != 0)
    def _acc():
        out_ref[0, 0] += s


def _tc_focal(x, t):
    grid = _TC_ROWS // _TC_BLOCK_ROWS
    return pl.pallas_call(
        _tc_focal_body,
        grid=(grid,),
        in_specs=[
            pl.BlockSpec((_TC_BLOCK_ROWS, _N_COLS), lambda i: (i + _TC_BLOCK_OFF, 0)),
            pl.BlockSpec((_TC_BLOCK_ROWS, _N_COLS), lambda i: (i + _TC_BLOCK_OFF, 0)),
        ],
        out_specs=pl.BlockSpec(memory_space=pltpu.SMEM),
        out_shape=jax.ShapeDtypeStruct((1, 1), jnp.float32),
    )(x, t)


def kernel(inputs, targets):
    total = jnp.float32(0.0)
    if _SC_ROWS > 0:
        total = total + jnp.sum(_sc_focal(inputs, targets))
    if _TC_ROWS > 0:
        total = total + _tc_focal(inputs, targets)[0, 0]
    return total / _TOTAL
